# baseline (speedup 1.0000x reference)
import functools

import jax
import jax.numpy as jnp
from jax import lax
from jax.experimental import pallas as pl
from jax.experimental.pallas import tpu as pltpu
from jax.experimental.pallas import tpu_sc as plsc

L = 16
NW = 32
N_SAMPLES = 4096
N_SITES = 100
D = 128
SITES_PAD = 112
N_CHUNKS = SITES_PAD // L
SPW = N_SAMPLES // NW
ROWS_PW = SPW * N_SITES
GCHUNK = 128
N_GCHUNKS = ROWS_PW // GCHUNK


def _vfull(val):
    return jnp.full((L,), val, jnp.int32)


def _bcast_last(cs):
    return plsc.cummax(lax.rev(cs, (0,)))


NBUF = 4


def _sc_body(x_hbm, table_hbm, out_hbm, x_v, idx2d,
             rows0, rows1, rows2, rows3,
             gsem0, gsem1, gsem2, gsem3,
             wsem0, wsem1, wsem2, wsem3):
    rows = (rows0, rows1, rows2, rows3)
    gsems = (gsem0, gsem1, gsem2, gsem3)
    wsems = (wsem0, wsem1, wsem2, wsem3)
    wid = lax.axis_index("s") * 2 + lax.axis_index("c")
    base_s = wid * SPW

    pltpu.sync_copy(x_hbm.at[pl.ds(base_s, SPW)], x_v)

    iota = lax.iota(jnp.int32, L)
    ones_v = _vfull(1)
    zeros_v = _vfull(0)
    negones_v = _vfull(-1)

    def build_sample(smp, smp_off):
        n_up = zeros_v
        for c in range(N_CHUNKS):
            v = x_v[smp, pl.ds(c * L, L)]
            upi = jnp.where(v == ones_v, ones_v, zeros_v)
            n_up = n_up + _bcast_last(plsc.cumsum(upi))
        carry = zeros_v
        for c in range(N_CHUNKS):
            v = x_v[smp, pl.ds(c * L, L)]
            up = v == ones_v
            dn = v == negones_v
            upi = jnp.where(up, ones_v, zeros_v)
            cs = plsc.cumsum(upi)
            up_ex = carry + cs - upi
            i_loc = iota + _vfull(c * L)
            p = jnp.where(up, up_ex, n_up + i_loc - up_ex)
            src = jnp.where(dn, i_loc + _vfull(N_SITES), i_loc)
            dest = smp_off + p
            row = lax.shift_right_logical(dest, _vfull(7))
            col = dest & _vfull(GCHUNK - 1)
            plsc.store_scatter(idx2d, [row, col], src, mask=up | dn)
            carry = carry + _bcast_last(cs)
        return smp_off + _vfull(N_SITES)

    lax.fori_loop(0, SPW, build_sample, zeros_v)

    out_base = wid * ROWS_PW

    def gather(k, b):
        pltpu.async_copy(table_hbm.at[idx2d.at[k]], rows[b], gsems[b])

    def wait_gather(b):
        pltpu.make_async_copy(table_hbm.at[idx2d.at[0]], rows[b], gsems[b]).wait()

    def write(k, b):
        pltpu.async_copy(
            rows[b], out_hbm.at[pl.ds(out_base + k * GCHUNK, GCHUNK)], wsems[b])

    def wait_write(b):
        pltpu.make_async_copy(
            rows[b], out_hbm.at[pl.ds(out_base, GCHUNK)], wsems[b]).wait()

    for b in range(NBUF - 1):
        gather(b, b)

    def move(g, _):
        k0 = g * NBUF
        for b in range(NBUF):
            k = k0 + b
            bg = (b + NBUF - 1) % NBUF
            kg = k + NBUF - 1
            wait_gather(b)
            write(k, b)
            if b == 0:
                @pl.when(g >= 1)
                def _():
                    wait_write(bg)
                gather(kg, bg)
            else:
                @pl.when(kg < N_GCHUNKS)
                def _():
                    wait_write(bg)
                    gather(kg, bg)
        return 0

    lax.fori_loop(0, N_GCHUNKS // NBUF, move, 0)

    for b in range(NBUF):
        wait_write(b)


_sc_kernel = functools.partial(
    pl.kernel,
    out_type=jax.ShapeDtypeStruct((N_SAMPLES * N_SITES, D), jnp.float32),
    mesh=plsc.VectorSubcoreMesh(core_axis_name="c", subcore_axis_name="s"),
    compiler_params=pltpu.CompilerParams(needs_layout_passes=False),
    scratch_types=[
        pltpu.VMEM((SPW, SITES_PAD), jnp.int32),
        pltpu.VMEM((N_GCHUNKS, GCHUNK), jnp.int32),
    ]
    + [pltpu.VMEM((GCHUNK, D), jnp.float32) for _ in range(NBUF)]
    + [pltpu.SemaphoreType.DMA for _ in range(2 * NBUF)],
)(_sc_body)


def kernel(x, orbitals_mf, orbitals_hf):
    n_samples, n_sites = x.shape
    assert (n_samples, n_sites) == (N_SAMPLES, N_SITES)
    table = jnp.concatenate([orbitals_mf, orbitals_hf], axis=1)
    xp = jnp.pad(x.astype(jnp.int32), ((0, 0), (0, SITES_PAD - n_sites)))
    out = _sc_kernel(xp, table)
    return out.reshape(n_samples, n_sites, D)


# speedup vs baseline: 1.3228x; 1.3228x over previous
import functools

import jax
import jax.numpy as jnp
from jax import lax
from jax.experimental import pallas as pl
from jax.experimental.pallas import tpu as pltpu
from jax.experimental.pallas import tpu_sc as plsc

L = 16
NW = 32
N_SAMPLES = 4096
N_SITES = 100
D = 128
SITES_PAD = 128
N_CHUNKS = SITES_PAD // L
SPW = N_SAMPLES // NW
IDX_STRIDE = 104
NBUF = 4


def _vfull(val):
    return jnp.full((L,), val, jnp.int32)


def _bcast_last(cs):
    return plsc.cummax(lax.rev(cs, (0,)))


def _sc_body(x_hbm, table_hbm, out_hbm, x_v, idx1d,
             rows0, rows1, rows2, rows3,
             gsem0, gsem1, gsem2, gsem3,
             wsem0, wsem1, wsem2, wsem3):
    rows = (rows0, rows1, rows2, rows3)
    gsems = (gsem0, gsem1, gsem2, gsem3)
    wsems = (wsem0, wsem1, wsem2, wsem3)

    wid = lax.axis_index("s") * 2 + lax.axis_index("c")
    base_s = wid * SPW

    pltpu.sync_copy(x_hbm.at[pl.ds(base_s, SPW)], x_v)

    iota = lax.iota(jnp.int32, L)
    ones_v = _vfull(1)
    zeros_v = _vfull(0)
    negones_v = _vfull(-1)

    def build_sample(smp, smp_off):
        n_up = zeros_v
        for c in range(N_CHUNKS):
            v = x_v[smp, pl.ds(c * L, L)]
            upi = jnp.where(v == ones_v, ones_v, zeros_v)
            n_up = n_up + _bcast_last(plsc.cumsum(upi))
        carry = zeros_v
        for c in range(N_CHUNKS):
            v = x_v[smp, pl.ds(c * L, L)]
            up = v == ones_v
            dn = v == negones_v
            upi = jnp.where(up, ones_v, zeros_v)
            cs = plsc.cumsum(upi)
            up_ex = carry + cs - upi
            i_loc = iota + _vfull(c * L)
            p = jnp.where(up, up_ex, n_up + i_loc - up_ex)
            src = jnp.where(dn, i_loc + _vfull(N_SITES), i_loc)
            plsc.store_scatter(idx1d, [smp_off + p], src, mask=up | dn)
            carry = carry + _bcast_last(cs)
        return smp_off + _vfull(IDX_STRIDE)

    lax.fori_loop(0, SPW, build_sample, zeros_v)

    def gather(k, b):
        pltpu.async_copy(
            table_hbm.at[idx1d.at[pl.ds(k * IDX_STRIDE, N_SITES)]],
            rows[b].at[pl.ds(0, N_SITES)], gsems[b])

    def wait_gather(b):
        pltpu.make_async_copy(
            table_hbm.at[idx1d.at[pl.ds(0, N_SITES)]],
            rows[b].at[pl.ds(0, N_SITES)], gsems[b]).wait()

    def write(k, b):
        pltpu.async_copy(
            rows[b].at[pl.ds(0, N_SITES)], out_hbm.at[base_s + k], wsems[b])

    def wait_write(b):
        pltpu.make_async_copy(
            rows[b].at[pl.ds(0, N_SITES)], out_hbm.at[base_s], wsems[b]).wait()

    for b in range(NBUF - 1):
        gather(b, b)

    def move(g, _):
        k0 = g * NBUF
        for b in range(NBUF):
            k = k0 + b
            bg = (b + NBUF - 1) % NBUF
            kg = k + NBUF - 1
            wait_gather(b)
            write(k, b)
            if b == 0:
                @pl.when(g >= 1)
                def _():
                    wait_write(bg)
                gather(kg, bg)
            else:
                @pl.when(kg < SPW)
                def _():
                    wait_write(bg)
                    gather(kg, bg)
        return 0

    lax.fori_loop(0, SPW // NBUF, move, 0)

    for b in range(NBUF):
        wait_write(b)


_sc_kernel = functools.partial(
    pl.kernel,
    out_type=jax.ShapeDtypeStruct((N_SAMPLES, N_SITES, D), jnp.float32),
    mesh=plsc.VectorSubcoreMesh(core_axis_name="c", subcore_axis_name="s"),
    compiler_params=pltpu.CompilerParams(
        needs_layout_passes=False, use_tc_tiling_on_sc=True),
    scratch_types=[
        pltpu.VMEM((SPW, SITES_PAD), jnp.int32),
        pltpu.VMEM((SPW * IDX_STRIDE,), jnp.int32),
    ]
    + [pltpu.VMEM((IDX_STRIDE, D), jnp.float32) for _ in range(NBUF)]
    + [pltpu.SemaphoreType.DMA for _ in range(2 * NBUF)],
)(_sc_body)


def kernel(x, orbitals_mf, orbitals_hf):
    n_samples, n_sites = x.shape
    assert (n_samples, n_sites) == (N_SAMPLES, N_SITES)
    table = jnp.concatenate([orbitals_mf, orbitals_hf], axis=1)
    xp = jnp.pad(x.astype(jnp.int32), ((0, 0), (0, SITES_PAD - n_sites)))
    return _sc_kernel(xp, table)


# speedup vs baseline: 1.3560x; 1.0250x over previous
import functools

import jax
import jax.numpy as jnp
from jax import lax
from jax.experimental import pallas as pl
from jax.experimental.pallas import tpu as pltpu
from jax.experimental.pallas import tpu_sc as plsc

L = 16
NW = 32
N_SAMPLES = 4096
N_SITES = 100
D = 128
SITES_PAD = 128
N_CHUNKS = SITES_PAD // L
SPW = N_SAMPLES // NW
IDX_STRIDE = 104
NBUF = 4


def _vfull(val):
    return jnp.full((L,), val, jnp.int32)


def _bcast_last(cs):
    return plsc.cummax(lax.rev(cs, (0,)))


def _sc_body(x_hbm, table_hbm, out_hbm, x_v, idx1d,
             rows0, rows1, rows2, rows3,
             gsem0, gsem1, gsem2, gsem3,
             wsem0, wsem1, wsem2, wsem3):
    rows = (rows0, rows1, rows2, rows3)
    gsems = (gsem0, gsem1, gsem2, gsem3)
    wsems = (wsem0, wsem1, wsem2, wsem3)

    wid = lax.axis_index("s") * 2 + lax.axis_index("c")
    base_s = wid * SPW

    pltpu.sync_copy(x_hbm.at[pl.ds(base_s, SPW)], x_v)

    iota = lax.iota(jnp.int32, L)
    ones_v = _vfull(1)
    zeros_v = _vfull(0)
    negones_v = _vfull(-1)

    def build_sample(smp, smp_off):
        n_up = zeros_v
        for c in range(N_CHUNKS):
            v = x_v[smp, pl.ds(c * L, L)]
            upi = jnp.where(v == ones_v, ones_v, zeros_v)
            n_up = n_up + _bcast_last(plsc.cumsum(upi))
        carry = zeros_v
        for c in range(N_CHUNKS):
            v = x_v[smp, pl.ds(c * L, L)]
            up = v == ones_v
            dn = v == negones_v
            upi = jnp.where(up, ones_v, zeros_v)
            cs = plsc.cumsum(upi)
            up_ex = carry + cs - upi
            i_loc = iota + _vfull(c * L)
            p = jnp.where(up, up_ex, n_up + i_loc - up_ex)
            src = jnp.where(dn, i_loc + _vfull(N_SITES), i_loc)
            plsc.store_scatter(idx1d, [smp_off + p], src, mask=up | dn)
            carry = carry + _bcast_last(cs)

    def gather(k, b):
        pltpu.async_copy(
            table_hbm.at[idx1d.at[pl.ds(k * IDX_STRIDE, N_SITES)]],
            rows[b].at[pl.ds(0, N_SITES)], gsems[b])

    def wait_gather(b):
        pltpu.make_async_copy(
            table_hbm.at[idx1d.at[pl.ds(0, N_SITES)]],
            rows[b].at[pl.ds(0, N_SITES)], gsems[b]).wait()

    def write(k, b):
        pltpu.async_copy(
            rows[b].at[pl.ds(0, N_SITES)], out_hbm.at[base_s + k], wsems[b])

    def wait_write(b):
        pltpu.make_async_copy(
            rows[b].at[pl.ds(0, N_SITES)], out_hbm.at[base_s], wsems[b]).wait()

    for b in range(NBUF - 1):
        build_sample(b, _vfull(b * IDX_STRIDE))
        gather(b, b)

    def move(g, off0):
        k0 = g * NBUF
        for b in range(NBUF):
            k = k0 + b
            bg = (b + NBUF - 1) % NBUF
            kg = k + NBUF - 1
            off_kg = off0 + _vfull((b + NBUF - 1) * IDX_STRIDE)
            wait_gather(b)
            write(k, b)
            if b == 0:
                build_sample(kg, off_kg)
                @pl.when(g >= 1)
                def _():
                    wait_write(bg)
                gather(kg, bg)
            else:
                @pl.when(kg < SPW)
                def _():
                    build_sample(kg, off_kg)
                    wait_write(bg)
                    gather(kg, bg)
        return off0 + _vfull(NBUF * IDX_STRIDE)

    lax.fori_loop(0, SPW // NBUF, move, zeros_v)

    for b in range(NBUF):
        wait_write(b)


_sc_kernel = functools.partial(
    pl.kernel,
    out_type=jax.ShapeDtypeStruct((N_SAMPLES, N_SITES, D), jnp.float32),
    mesh=plsc.VectorSubcoreMesh(core_axis_name="c", subcore_axis_name="s"),
    compiler_params=pltpu.CompilerParams(
        needs_layout_passes=False, use_tc_tiling_on_sc=True),
    scratch_types=[
        pltpu.VMEM((SPW, SITES_PAD), jnp.int32),
        pltpu.VMEM((SPW * IDX_STRIDE,), jnp.int32),
    ]
    + [pltpu.VMEM((IDX_STRIDE, D), jnp.float32) for _ in range(NBUF)]
    + [pltpu.SemaphoreType.DMA for _ in range(2 * NBUF)],
)(_sc_body)


def kernel(x, orbitals_mf, orbitals_hf):
    n_samples, n_sites = x.shape
    assert (n_samples, n_sites) == (N_SAMPLES, N_SITES)
    table = jnp.concatenate([orbitals_mf, orbitals_hf], axis=1)
    xp = jnp.pad(x.astype(jnp.int32), ((0, 0), (0, SITES_PAD - n_sites)))
    return _sc_kernel(xp, table)
